# Initial kernel scaffold; baseline (speedup 1.0000x reference)
import jax
import jax.numpy as jnp
from jax.experimental import pallas as pl


def kernel(text, emb_table, fc_w, fc_b):
    raise NotImplementedError("write your pallas kernel here")


# speedup vs baseline: 2.1929x; 2.1929x over previous
import functools

import jax
import jax.numpy as jnp
from jax import lax
from jax.experimental import pallas as pl
from jax.experimental.pallas import tpu as pltpu
from jax.experimental.pallas import tpu_sc as plsc

B = 4096
H = 200
D = 32
NC, NS = 2, 16
NW = NC * NS
BAGS_PER_W = B // NW
CHUNK_BAGS = 4
CHUNK_ROWS = CHUNK_BAGS * H
NCHUNK = BAGS_PER_W // CHUNK_BAGS
GATHER_W = 100
NGATHER = CHUNK_ROWS // GATHER_W

_mesh = plsc.VectorSubcoreMesh(core_axis_name="c", subcore_axis_name="s")


@functools.partial(
    pl.kernel,
    out_type=jax.ShapeDtypeStruct((B, D), jnp.float32),
    mesh=_mesh,
    scratch_types=[
        pltpu.VMEM((NGATHER, GATHER_W), jnp.int32),
        pltpu.VMEM((CHUNK_ROWS, D), jnp.float32),
        pltpu.VMEM((BAGS_PER_W, D), jnp.float32),
        pltpu.SemaphoreType.DMA,
    ],
    compiler_params=pltpu.CompilerParams(use_tc_tiling_on_sc=False),
)
def _bag_sums(text_hbm, table_hbm, out_hbm, idx_ref, rows_ref, sums_ref, sem):
    wid = lax.axis_index("c") * NS + lax.axis_index("s")
    zero = jnp.zeros((16,), jnp.float32)

    @pl.loop(0, NCHUNK)
    def _(c):
        row0 = wid * (BAGS_PER_W * 2) + c * (CHUNK_BAGS * 2)
        pltpu.sync_copy(text_hbm.at[pl.ds(row0, NGATHER)], idx_ref)
        cps = [
            pltpu.async_copy(
                table_hbm.at[idx_ref.at[j]],
                rows_ref.at[pl.ds(j * GATHER_W, GATHER_W)],
                sem,
            )
            for j in range(NGATHER)
        ]
        for cp in cps:
            cp.wait()
        for b in range(CHUNK_BAGS):
            def body(i, acc, _b=b):
                r = _b * H + i
                return (
                    acc[0] + rows_ref[r, pl.ds(0, 16)],
                    acc[1] + rows_ref[r, pl.ds(16, 16)],
                )
            a0, a1 = lax.fori_loop(0, H, body, (zero, zero), unroll=8)
            bag = c * CHUNK_BAGS + b
            sums_ref[bag, pl.ds(0, 16)] = a0
            sums_ref[bag, pl.ds(16, 16)] = a1

    pltpu.sync_copy(sums_ref, out_hbm.at[pl.ds(wid * BAGS_PER_W, BAGS_PER_W)])


def _linear_body(x_ref, w_ref, b_ref, o_ref):
    o_ref[...] = (
        jnp.dot(x_ref[...], w_ref[...], preferred_element_type=jnp.float32)
        * (1.0 / H)
        + b_ref[...]
    )


def _linear(sums, w_t, bias2d):
    return pl.pallas_call(
        _linear_body,
        out_shape=jax.ShapeDtypeStruct((B, bias2d.shape[1]), jnp.float32),
    )(sums, w_t, bias2d)


@jax.jit
def kernel(text, emb_table, fc_w, fc_b):
    text2d = text.reshape(B * 2, H // 2).astype(jnp.int32)
    sums = _bag_sums(text2d, emb_table)
    return _linear(sums, fc_w.T, fc_b.reshape(1, -1))
